# baseline (speedup 1.0000x reference)
import functools

import jax
import jax.numpy as jnp
from jax import lax
from jax.experimental import pallas as pl
from jax.experimental.pallas import tpu as pltpu
from jax.experimental.pallas import tpu_sc as plsc

N = 10000
E = 320000
IN_DIM = 128
H1, C1 = 8, 64
C2 = 32
OUT_DIM = 40

NPAD = 10112
ESL = E + N
EPAD = 331776
CH = 1152
SUB = CH // 128
RPT = NPAD // 16


def _tc_pre_body(x_ref, w1_ref, t1_ref, h1tab_ref, tab1_ref):
    h = jnp.dot(x_ref[...], w1_ref[...], preferred_element_type=jnp.float32)
    for hd in range(H1):
        h1tab_ref[hd] = h[:, C1 * hd:C1 * (hd + 1)]
        tab1_ref[hd] = jnp.dot(h, t1_ref[hd], preferred_element_type=jnp.float32)


def _tc_pre(xpad, W1, T1):
    BN = 2528
    return pl.pallas_call(
        _tc_pre_body,
        grid=(NPAD // BN,),
        in_specs=[
            pl.BlockSpec((BN, IN_DIM), lambda n: (n, 0)),
            pl.BlockSpec((IN_DIM, H1 * C1), lambda n: (0, 0)),
            pl.BlockSpec((H1, H1 * C1, 2), lambda n: (0, 0, 0)),
        ],
        out_specs=[
            pl.BlockSpec((H1, BN, C1), lambda n: (0, n, 0)),
            pl.BlockSpec((H1, BN, 2), lambda n: (0, n, 0)),
        ],
        out_shape=[
            jax.ShapeDtypeStruct((H1, NPAD, C1), jnp.float32),
            jax.ShapeDtypeStruct((H1, NPAD, 2), jnp.float32),
        ],
    )(xpad, W1, T1)


def _tc_mid_body(o1_ref, b1_ref, w2_ref, t2_ref, h2tab_ref, tab2_ref):
    cols = []
    for hd in range(H1):
        dat = o1_ref[hd]
        cols.append(dat[:, 0:C1] * (1.0 / (dat[:, C1:C1 + 1] + 1e-16)))
    x2 = jnp.concatenate(cols, axis=1) + b1_ref[...]
    x2 = jnp.maximum(x2, 0.0)
    h2 = jnp.dot(x2, w2_ref[...], preferred_element_type=jnp.float32)
    h2tab_ref[...] = h2
    tab2_ref[...] = jnp.dot(h2, t2_ref[...], preferred_element_type=jnp.float32)


def _tc_mid(out1, b1, W2, T2):
    BN = 2528
    return pl.pallas_call(
        _tc_mid_body,
        grid=(NPAD // BN,),
        in_specs=[
            pl.BlockSpec((H1, BN, 80), lambda n: (0, n, 0)),
            pl.BlockSpec((1, H1 * C1), lambda n: (0, 0)),
            pl.BlockSpec((H1 * C1, C2), lambda n: (0, 0)),
            pl.BlockSpec((C2, 2), lambda n: (0, 0)),
        ],
        out_specs=[
            pl.BlockSpec((BN, C2), lambda n: (n, 0)),
            pl.BlockSpec((BN, 2), lambda n: (n, 0)),
        ],
        out_shape=[
            jax.ShapeDtypeStruct((NPAD, C2), jnp.float32),
            jax.ShapeDtypeStruct((NPAD, 2), jnp.float32),
        ],
    )(out1, b1, W2, T2)


def _tc_post_body(o2_ref, b2_ref, fcw_ref, fcb_ref, emb_ref, log_ref):
    d = o2_ref[0, :, 0:C2] + o2_ref[1, :, 0:C2]
    den = o2_ref[0, :, C2:C2 + 1] + o2_ref[1, :, C2:C2 + 1]
    emb = jnp.maximum(d / (den + 1e-16) + b2_ref[...], 0.0)
    emb_ref[...] = emb
    log_ref[...] = (jnp.dot(emb, fcw_ref[...], preferred_element_type=jnp.float32)
                    + fcb_ref[...])


def _tc_post(out2, b2, fcW, fcb):
    BN = 2000
    return pl.pallas_call(
        _tc_post_body,
        grid=(N // BN,),
        in_specs=[
            pl.BlockSpec((2, BN, 48), lambda n: (0, n, 0)),
            pl.BlockSpec((1, C2), lambda n: (0, 0)),
            pl.BlockSpec((C2, OUT_DIM), lambda n: (0, 0)),
            pl.BlockSpec((1, OUT_DIM), lambda n: (0, 0)),
        ],
        out_specs=[
            pl.BlockSpec((BN, C2), lambda n: (n, 0)),
            pl.BlockSpec((BN, OUT_DIM), lambda n: (n, 0)),
        ],
        out_shape=[
            jax.ShapeDtypeStruct((N, C2), jnp.float32),
            jax.ShapeDtypeStruct((N, OUT_DIM), jnp.float32),
        ],
    )(out2, b2, fcW, fcb)


def _make_sc_gat(nvd, n_passes, split_edges):
    dcols = nvd * 16
    roww = dcols + 16
    ept = EPAD // (32 if split_edges else 16)
    nblk = ept // CH
    outr = (n_passes * 2 if not split_edges else 2) * NPAD

    mesh = plsc.VectorSubcoreMesh(core_axis_name="c", subcore_axis_name="s")
    iota16 = lambda: lax.broadcasted_iota(jnp.int32, (16,), 0)

    @functools.partial(
        pl.kernel,
        out_type=jax.ShapeDtypeStruct((outr, roww), jnp.float32),
        mesh=mesh,
        compiler_params=pltpu.CompilerParams(needs_layout_passes=False,
                                             use_tc_tiling_on_sc=False),
        scratch_types=[
            pltpu.VMEM((NPAD * 2,), jnp.float32),
            pltpu.VMEM((CH,), jnp.int32),
            pltpu.VMEM((CH,), jnp.int32),
            pltpu.VMEM((128,), jnp.int32),
            pltpu.VMEM((128,), jnp.int32),
            pltpu.VMEM((128,), jnp.float32),
            pltpu.VMEM((128, nvd * 16), jnp.float32),
            pltpu.VMEM((128, nvd * 16 + 16), jnp.float32),
            pltpu.VMEM((64, nvd * 16 + 16), jnp.float32),
            pltpu.VMEM_SHARED((NPAD, nvd * 16 + 16), jnp.float32),
        ],
    )
    def sc_kernel(tab_hbm, rows_hbm, src_hbm, dst_hbm, out_hbm,
                  tabv, srcv, dstv, idxv, dstiv, aeav, rowg, rows, zb, acc):
        c = lax.axis_index("c")
        s = lax.axis_index("s")

        def _zb(e, _):
            for j in range(roww // 16):
                zb[e, pl.ds(j * 16, 16)] = jnp.zeros((16,), jnp.float32)
            return 0
        lax.fori_loop(0, 64, _zb, 0)

        for gi in range(n_passes):
            if split_edges:
                tg = 0
                outbase = c * NPAD
                ebase = (c * 16 + s) * ept
            else:
                tg = n_passes * c + gi
                outbase = tg * NPAD
                ebase = s * ept

            r0 = s * RPT
            for k in range(RPT // 64):
                pltpu.sync_copy(zb, acc.at[pl.ds(r0 + k * 64, 64)])
            rem = RPT % 64
            if rem:
                pltpu.sync_copy(zb.at[pl.ds(0, rem)],
                                acc.at[pl.ds(r0 + (RPT // 64) * 64, rem)])

            pltpu.sync_copy(tab_hbm.at[pl.ds(tg * (NPAD * 2), NPAD * 2)], tabv)
            plsc.subcore_barrier()

            rowbase = tg * NPAD

            def blk_body(blk, _):
                eoff = ebase + blk * CH
                pltpu.sync_copy(src_hbm.at[pl.ds(eoff, CH)], srcv)
                pltpu.sync_copy(dst_hbm.at[pl.ds(eoff, CH)], dstv)

                def sub_body(sub, _):
                    def quad(q, _):
                        o = sub * 128 + q * 16
                        s16 = srcv[pl.ds(o, 16)]
                        d16 = dstv[pl.ds(o, 16)]
                        asl = plsc.load_gather(tabv, [s16 * 2])
                        adl = plsc.load_gather(tabv, [d16 * 2 + 1])
                        a = asl + adl
                        a = jnp.where(a > 0, a, a * 0.2)
                        aeav[pl.ds(q * 16, 16)] = jnp.exp(a)
                        idxv[pl.ds(q * 16, 16)] = rowbase + s16
                        dstiv[pl.ds(q * 16, 16)] = d16
                        return 0
                    lax.fori_loop(0, 8, quad, 0)

                    pltpu.sync_copy(rows_hbm.at[idxv], rowg)

                    def scale(q, _):
                        it = iota16()
                        av = aeav[pl.ds(q * 16, 16)]
                        for el in range(16):
                            e = q * 16 + el
                            sA = av[el]
                            for j in range(nvd):
                                rows[e, pl.ds(j * 16, 16)] = (
                                    rowg[e, pl.ds(j * 16, 16)] * sA)
                            rows[e, pl.ds(nvd * 16, 16)] = jnp.where(
                                it == 0, sA, 0.0)
                        return 0
                    lax.fori_loop(0, 8, scale, 0)

                    pltpu.sync_copy(rows, acc.at[dstiv], add=True)
                    return 0
                lax.fori_loop(0, SUB, sub_body, 0)
                return 0
            lax.fori_loop(0, nblk, blk_body, 0)

            plsc.subcore_barrier()
            pltpu.sync_copy(acc.at[pl.ds(r0, RPT)],
                            out_hbm.at[pl.ds(outbase + r0, RPT)])
            plsc.subcore_barrier()

    return sc_kernel


_sc_gat1 = _make_sc_gat(nvd=4, n_passes=4, split_edges=False)
_sc_gat2 = _make_sc_gat(nvd=2, n_passes=1, split_edges=True)


def kernel(x, edge_index, W1, att_src1, att_dst1, b1, W2, att_src2, att_dst2,
           b2, fcW, fcb):
    src = edge_index[0].astype(jnp.int32)
    dst = edge_index[1].astype(jnp.int32)
    loop = jnp.arange(N, dtype=jnp.int32)
    padv = jnp.full((EPAD - ESL,), N, dtype=jnp.int32)
    srcp = jnp.concatenate([src, loop, padv])
    dstp = jnp.concatenate([dst, loop, padv])

    xpad = jnp.concatenate(
        [x, jnp.zeros((NPAD - N, IN_DIM), jnp.float32)], axis=0)

    T1 = jnp.zeros((H1, H1 * C1, 2), jnp.float32)
    for hd in range(H1):
        T1 = T1.at[hd, hd * C1:(hd + 1) * C1, 0].set(att_src1[hd])
        T1 = T1.at[hd, hd * C1:(hd + 1) * C1, 1].set(att_dst1[hd])
    T2 = jnp.stack([att_src2[0], att_dst2[0]], axis=1)

    h1tab, tab1 = _tc_pre(xpad, W1, T1)
    out1 = _sc_gat1(tab1.reshape(H1 * NPAD * 2), h1tab.reshape(H1 * NPAD, C1),
                    srcp, dstp)
    h2tab, tab2 = _tc_mid(out1.reshape(H1, NPAD, 80), b1.reshape(1, H1 * C1),
                          W2, T2)
    out2 = _sc_gat2(tab2.reshape(NPAD * 2), h2tab, srcp, dstp)
    emb, logits = _tc_post(out2.reshape(2, NPAD, 48), b2.reshape(1, C2), fcW,
                           fcb.reshape(1, OUT_DIM))
    return (emb, logits)


# speedup vs baseline: 1.1975x; 1.1975x over previous
import functools

import jax
import jax.numpy as jnp
from jax import lax
from jax.experimental import pallas as pl
from jax.experimental.pallas import tpu as pltpu
from jax.experimental.pallas import tpu_sc as plsc

N = 10000
E = 320000
IN_DIM = 128
H1, C1 = 8, 64
C2 = 32
OUT_DIM = 40

NPAD = 10112
ESL = E + N
EPAD = 331776
CH = 1152
SUB = CH // 128
RPT = NPAD // 16


def _tc_pre_body(x_ref, w1_ref, t1_ref, h1tab_ref, tab1_ref):
    h = jnp.dot(x_ref[...], w1_ref[...], preferred_element_type=jnp.float32)
    for hd in range(H1):
        h1tab_ref[hd] = h[:, C1 * hd:C1 * (hd + 1)]
        tab1_ref[hd] = jnp.dot(h, t1_ref[hd], preferred_element_type=jnp.float32)


def _tc_pre(xpad, W1, T1):
    BN = 2528
    return pl.pallas_call(
        _tc_pre_body,
        grid=(NPAD // BN,),
        in_specs=[
            pl.BlockSpec((BN, IN_DIM), lambda n: (n, 0)),
            pl.BlockSpec((IN_DIM, H1 * C1), lambda n: (0, 0)),
            pl.BlockSpec((H1, H1 * C1, 2), lambda n: (0, 0, 0)),
        ],
        out_specs=[
            pl.BlockSpec((H1, BN, C1), lambda n: (0, n, 0)),
            pl.BlockSpec((H1, BN, 2), lambda n: (0, n, 0)),
        ],
        out_shape=[
            jax.ShapeDtypeStruct((H1, NPAD, C1), jnp.float32),
            jax.ShapeDtypeStruct((H1, NPAD, 2), jnp.float32),
        ],
    )(xpad, W1, T1)


def _tc_mid_body(o1d_ref, o1w_ref, b1_ref, w2_ref, t2_ref, h2tab_ref,
                 tab2_ref):
    cols = []
    for hd in range(H1):
        den = o1w_ref[hd][:, 0:1]
        cols.append(o1d_ref[hd] * (1.0 / (den + 1e-16)))
    x2 = jnp.concatenate(cols, axis=1) + b1_ref[...]
    x2 = jnp.maximum(x2, 0.0)
    h2 = jnp.dot(x2, w2_ref[...], preferred_element_type=jnp.float32)
    h2tab_ref[...] = h2
    tab2_ref[...] = jnp.dot(h2, t2_ref[...], preferred_element_type=jnp.float32)


def _tc_mid(out1d, out1w, b1, W2, T2):
    BN = 2528
    return pl.pallas_call(
        _tc_mid_body,
        grid=(NPAD // BN,),
        in_specs=[
            pl.BlockSpec((H1, BN, C1), lambda n: (0, n, 0)),
            pl.BlockSpec((H1, BN, 16), lambda n: (0, n, 0)),
            pl.BlockSpec((1, H1 * C1), lambda n: (0, 0)),
            pl.BlockSpec((H1 * C1, C2), lambda n: (0, 0)),
            pl.BlockSpec((C2, 2), lambda n: (0, 0)),
        ],
        out_specs=[
            pl.BlockSpec((BN, C2), lambda n: (n, 0)),
            pl.BlockSpec((BN, 2), lambda n: (n, 0)),
        ],
        out_shape=[
            jax.ShapeDtypeStruct((NPAD, C2), jnp.float32),
            jax.ShapeDtypeStruct((NPAD, 2), jnp.float32),
        ],
    )(out1d, out1w, b1, W2, T2)


def _tc_post_body(o2d_ref, o2w_ref, b2_ref, fcw_ref, fcb_ref, emb_ref,
                  log_ref):
    d = o2d_ref[0] + o2d_ref[1]
    den = o2w_ref[0][:, 0:1] + o2w_ref[1][:, 0:1]
    emb = jnp.maximum(d / (den + 1e-16) + b2_ref[...], 0.0)
    emb_ref[...] = emb
    log_ref[...] = (jnp.dot(emb, fcw_ref[...], preferred_element_type=jnp.float32)
                    + fcb_ref[...])


def _tc_post(out2d, out2w, b2, fcW, fcb):
    BN = 2000
    return pl.pallas_call(
        _tc_post_body,
        grid=(N // BN,),
        in_specs=[
            pl.BlockSpec((2, BN, C2), lambda n: (0, n, 0)),
            pl.BlockSpec((2, BN, 16), lambda n: (0, n, 0)),
            pl.BlockSpec((1, C2), lambda n: (0, 0)),
            pl.BlockSpec((C2, OUT_DIM), lambda n: (0, 0)),
            pl.BlockSpec((1, OUT_DIM), lambda n: (0, 0)),
        ],
        out_specs=[
            pl.BlockSpec((BN, C2), lambda n: (n, 0)),
            pl.BlockSpec((BN, OUT_DIM), lambda n: (n, 0)),
        ],
        out_shape=[
            jax.ShapeDtypeStruct((N, C2), jnp.float32),
            jax.ShapeDtypeStruct((N, OUT_DIM), jnp.float32),
        ],
    )(out2d, out2w, b2, fcW, fcb)


def _make_sc_gat(nvd, n_passes, split_edges):
    dcols = nvd * 16
    ept = EPAD // (32 if split_edges else 16)
    nblk = ept // CH
    ngrp = n_passes * 2 if not split_edges else 2
    outr = ngrp * NPAD

    mesh = plsc.VectorSubcoreMesh(core_axis_name="c", subcore_axis_name="s")
    iota16 = lambda: lax.broadcasted_iota(jnp.int32, (16,), 0)

    @functools.partial(
        pl.kernel,
        out_type=[
            jax.ShapeDtypeStruct((outr, dcols), jnp.float32),
            jax.ShapeDtypeStruct((outr, 16), jnp.float32),
        ],
        mesh=mesh,
        compiler_params=pltpu.CompilerParams(needs_layout_passes=False,
                                             use_tc_tiling_on_sc=False),
        scratch_types=[
            pltpu.VMEM((NPAD * 2,), jnp.float32),
            pltpu.VMEM((CH,), jnp.int32),
            pltpu.VMEM((CH,), jnp.int32),
            pltpu.VMEM((128,), jnp.int32),
            pltpu.VMEM((128,), jnp.int32),
            pltpu.VMEM((128,), jnp.float32),
            pltpu.VMEM((128, nvd * 16), jnp.float32),
            pltpu.VMEM((128, 16), jnp.float32),
            pltpu.VMEM((64, nvd * 16), jnp.float32),
            pltpu.VMEM((64, 16), jnp.float32),
            pltpu.VMEM_SHARED((NPAD, nvd * 16), jnp.float32),
            pltpu.VMEM_SHARED((NPAD, nvd * 16), jnp.float32),
            pltpu.VMEM_SHARED((NPAD, 16), jnp.float32),
        ],
    )
    def sc_kernel(tab_hbm, rows_hbm, src_hbm, dst_hbm, outd_hbm, outw_hbm,
                  tabv, srcv, dstv, idxv, dstiv, aeav, rowg, wden, zbd, zbw,
                  rows_sp, accd, accw):
        c = lax.axis_index("c")
        s = lax.axis_index("s")

        def _zb(e, _):
            for j in range(nvd):
                zbd[e, pl.ds(j * 16, 16)] = jnp.zeros((16,), jnp.float32)
            zbw[e] = jnp.zeros((16,), jnp.float32)
            return 0
        lax.fori_loop(0, 64, _zb, 0)

        for gi in range(n_passes):
            if split_edges:
                tg = 0
                outbase = c * NPAD
                ebase = (c * 16 + s) * ept
            else:
                tg = n_passes * c + gi
                outbase = tg * NPAD
                ebase = s * ept

            r0 = s * RPT
            for k in range(RPT // 64):
                pltpu.sync_copy(zbd, accd.at[pl.ds(r0 + k * 64, 64)])
                pltpu.sync_copy(zbw, accw.at[pl.ds(r0 + k * 64, 64)])
            rem = RPT % 64
            if rem:
                ro = r0 + (RPT // 64) * 64
                pltpu.sync_copy(zbd.at[pl.ds(0, rem)],
                                accd.at[pl.ds(ro, rem)])
                pltpu.sync_copy(zbw.at[pl.ds(0, rem)],
                                accw.at[pl.ds(ro, rem)])

            rowbase = tg * NPAD
            pltpu.sync_copy(tab_hbm.at[pl.ds(tg * (NPAD * 2), NPAD * 2)], tabv)
            pltpu.sync_copy(rows_hbm.at[pl.ds(rowbase + r0, RPT)],
                            rows_sp.at[pl.ds(r0, RPT)])
            plsc.subcore_barrier()

            def blk_body(blk, _):
                eoff = ebase + blk * CH
                pltpu.sync_copy(src_hbm.at[pl.ds(eoff, CH)], srcv)
                pltpu.sync_copy(dst_hbm.at[pl.ds(eoff, CH)], dstv)

                def sub_body(sub, _):
                    def quad(q, _):
                        o = sub * 128 + q * 16
                        s16 = srcv[pl.ds(o, 16)]
                        d16 = dstv[pl.ds(o, 16)]
                        asl = plsc.load_gather(tabv, [s16 * 2])
                        adl = plsc.load_gather(tabv, [d16 * 2 + 1])
                        a = asl + adl
                        a = jnp.where(a > 0, a, a * 0.2)
                        aeav[pl.ds(q * 16, 16)] = jnp.exp(a)
                        idxv[pl.ds(q * 16, 16)] = s16
                        dstiv[pl.ds(q * 16, 16)] = d16
                        return 0
                    lax.fori_loop(0, 8, quad, 0)

                    pltpu.sync_copy(rows_sp.at[idxv], rowg)

                    def scale(q, _):
                        it = iota16()
                        av = aeav[pl.ds(q * 16, 16)]
                        for el in range(16):
                            e = q * 16 + el
                            sA = av[el]
                            for j in range(nvd):
                                rowg[e, pl.ds(j * 16, 16)] = (
                                    rowg[e, pl.ds(j * 16, 16)] * sA)
                            wden[e] = jnp.where(it == 0, sA, 0.0)
                        return 0
                    lax.fori_loop(0, 8, scale, 0)

                    pltpu.sync_copy(rowg, accd.at[dstiv], add=True)
                    pltpu.sync_copy(wden, accw.at[dstiv], add=True)
                    return 0
                lax.fori_loop(0, SUB, sub_body, 0)
                return 0
            lax.fori_loop(0, nblk, blk_body, 0)

            plsc.subcore_barrier()
            pltpu.sync_copy(accd.at[pl.ds(r0, RPT)],
                            outd_hbm.at[pl.ds(outbase + r0, RPT)])
            pltpu.sync_copy(accw.at[pl.ds(r0, RPT)],
                            outw_hbm.at[pl.ds(outbase + r0, RPT)])
            plsc.subcore_barrier()

    return sc_kernel


_sc_gat1 = _make_sc_gat(nvd=4, n_passes=4, split_edges=False)
_sc_gat2 = _make_sc_gat(nvd=2, n_passes=1, split_edges=True)


def kernel(x, edge_index, W1, att_src1, att_dst1, b1, W2, att_src2, att_dst2,
           b2, fcW, fcb):
    src = edge_index[0].astype(jnp.int32)
    dst = edge_index[1].astype(jnp.int32)
    loop = jnp.arange(N, dtype=jnp.int32)
    padv = jnp.full((EPAD - ESL,), N, dtype=jnp.int32)
    srcp = jnp.concatenate([src, loop, padv])
    dstp = jnp.concatenate([dst, loop, padv])

    xpad = jnp.concatenate(
        [x, jnp.zeros((NPAD - N, IN_DIM), jnp.float32)], axis=0)

    T1 = jnp.zeros((H1, H1 * C1, 2), jnp.float32)
    for hd in range(H1):
        T1 = T1.at[hd, hd * C1:(hd + 1) * C1, 0].set(att_src1[hd])
        T1 = T1.at[hd, hd * C1:(hd + 1) * C1, 1].set(att_dst1[hd])
    T2 = jnp.stack([att_src2[0], att_dst2[0]], axis=1)

    h1tab, tab1 = _tc_pre(xpad, W1, T1)
    out1d, out1w = _sc_gat1(tab1.reshape(H1 * NPAD * 2),
                            h1tab.reshape(H1 * NPAD, C1), srcp, dstp)
    h2tab, tab2 = _tc_mid(out1d.reshape(H1, NPAD, C1),
                          out1w.reshape(H1, NPAD, 16),
                          b1.reshape(1, H1 * C1), W2, T2)
    out2d, out2w = _sc_gat2(tab2.reshape(NPAD * 2), h2tab, srcp, dstp)
    emb, logits = _tc_post(out2d.reshape(2, NPAD, C2),
                           out2w.reshape(2, NPAD, 16),
                           b2.reshape(1, C2), fcW, fcb.reshape(1, OUT_DIM))
    return (emb, logits)


# speedup vs baseline: 1.2912x; 1.0782x over previous
import functools

import jax
import jax.numpy as jnp
from jax import lax
from jax.experimental import pallas as pl
from jax.experimental.pallas import tpu as pltpu
from jax.experimental.pallas import tpu_sc as plsc

N = 10000
E = 320000
IN_DIM = 128
H1, C1 = 8, 64
C2 = 32
OUT_DIM = 40

NPAD = 10112
ESL = E + N
EPAD = 331776
CH = 1152
SUB = CH // 128
RPT = NPAD // 16


def _tc_pre_body(x_ref, w1_ref, t1_ref, h1tab_ref, tab1_ref):
    h = jnp.dot(x_ref[...], w1_ref[...], preferred_element_type=jnp.float32)
    for hd in range(H1):
        h1tab_ref[hd] = h[:, C1 * hd:C1 * (hd + 1)]
        tab1_ref[hd] = jnp.dot(h, t1_ref[hd], preferred_element_type=jnp.float32)


def _tc_pre(xpad, W1, T1):
    BN = 2528
    return pl.pallas_call(
        _tc_pre_body,
        grid=(NPAD // BN,),
        in_specs=[
            pl.BlockSpec((BN, IN_DIM), lambda n: (n, 0)),
            pl.BlockSpec((IN_DIM, H1 * C1), lambda n: (0, 0)),
            pl.BlockSpec((H1, H1 * C1, 2), lambda n: (0, 0, 0)),
        ],
        out_specs=[
            pl.BlockSpec((H1, BN, C1), lambda n: (0, n, 0)),
            pl.BlockSpec((H1, BN, 2), lambda n: (0, n, 0)),
        ],
        out_shape=[
            jax.ShapeDtypeStruct((H1, NPAD, C1), jnp.float32),
            jax.ShapeDtypeStruct((H1, NPAD, 2), jnp.float32),
        ],
    )(xpad, W1, T1)


def _tc_mid_body(o1d_ref, o1w_ref, b1_ref, w2_ref, t2_ref, h2tab_ref,
                 tab2_ref):
    cols = []
    for hd in range(H1):
        den = o1w_ref[hd][:, 0:1]
        cols.append(o1d_ref[hd] * (1.0 / (den + 1e-16)))
    x2 = jnp.concatenate(cols, axis=1) + b1_ref[...]
    x2 = jnp.maximum(x2, 0.0)
    h2 = jnp.dot(x2, w2_ref[...], preferred_element_type=jnp.float32)
    h2tab_ref[...] = h2
    tab2_ref[...] = jnp.dot(h2, t2_ref[...], preferred_element_type=jnp.float32)


def _tc_mid(out1d, out1w, b1, W2, T2):
    BN = 2528
    return pl.pallas_call(
        _tc_mid_body,
        grid=(NPAD // BN,),
        in_specs=[
            pl.BlockSpec((H1, BN, C1), lambda n: (0, n, 0)),
            pl.BlockSpec((H1, BN, 16), lambda n: (0, n, 0)),
            pl.BlockSpec((1, H1 * C1), lambda n: (0, 0)),
            pl.BlockSpec((H1 * C1, C2), lambda n: (0, 0)),
            pl.BlockSpec((C2, 2), lambda n: (0, 0)),
        ],
        out_specs=[
            pl.BlockSpec((BN, C2), lambda n: (n, 0)),
            pl.BlockSpec((BN, 2), lambda n: (n, 0)),
        ],
        out_shape=[
            jax.ShapeDtypeStruct((NPAD, C2), jnp.float32),
            jax.ShapeDtypeStruct((NPAD, 2), jnp.float32),
        ],
    )(out1d, out1w, b1, W2, T2)


def _tc_post_body(o2d_ref, o2w_ref, b2_ref, fcw_ref, fcb_ref, emb_ref,
                  log_ref):
    d = o2d_ref[0] + o2d_ref[1]
    den = o2w_ref[0][:, 0:1] + o2w_ref[1][:, 0:1]
    emb = jnp.maximum(d / (den + 1e-16) + b2_ref[...], 0.0)
    emb_ref[...] = emb
    log_ref[...] = (jnp.dot(emb, fcw_ref[...], preferred_element_type=jnp.float32)
                    + fcb_ref[...])


def _tc_post(out2d, out2w, b2, fcW, fcb):
    BN = 2000
    return pl.pallas_call(
        _tc_post_body,
        grid=(N // BN,),
        in_specs=[
            pl.BlockSpec((2, BN, C2), lambda n: (0, n, 0)),
            pl.BlockSpec((2, BN, 16), lambda n: (0, n, 0)),
            pl.BlockSpec((1, C2), lambda n: (0, 0)),
            pl.BlockSpec((C2, OUT_DIM), lambda n: (0, 0)),
            pl.BlockSpec((1, OUT_DIM), lambda n: (0, 0)),
        ],
        out_specs=[
            pl.BlockSpec((BN, C2), lambda n: (n, 0)),
            pl.BlockSpec((BN, OUT_DIM), lambda n: (n, 0)),
        ],
        out_shape=[
            jax.ShapeDtypeStruct((N, C2), jnp.float32),
            jax.ShapeDtypeStruct((N, OUT_DIM), jnp.float32),
        ],
    )(out2d, out2w, b2, fcW, fcb)


def _make_sc_gat(nvd, n_passes, split_edges):
    dcols = nvd * 16
    ept = EPAD // (32 if split_edges else 16)
    nblk = ept // CH
    ngrp = n_passes * 2 if not split_edges else 2
    outr = ngrp * NPAD
    HB = CH // 64

    mesh = plsc.VectorSubcoreMesh(core_axis_name="c", subcore_axis_name="s")
    iota16 = lambda: lax.broadcasted_iota(jnp.int32, (16,), 0)

    @functools.partial(
        pl.kernel,
        out_type=[
            jax.ShapeDtypeStruct((outr, dcols), jnp.float32),
            jax.ShapeDtypeStruct((outr, 16), jnp.float32),
        ],
        mesh=mesh,
        compiler_params=pltpu.CompilerParams(needs_layout_passes=False,
                                             use_tc_tiling_on_sc=False),
        scratch_types=[
            pltpu.VMEM((NPAD * 2,), jnp.float32),
            pltpu.VMEM((CH,), jnp.int32),
            pltpu.VMEM((CH,), jnp.int32),
            pltpu.VMEM((64,), jnp.int32),
            pltpu.VMEM((64,), jnp.int32),
            pltpu.VMEM((64,), jnp.int32),
            pltpu.VMEM((64,), jnp.int32),
            pltpu.VMEM((64,), jnp.float32),
            pltpu.VMEM((64,), jnp.float32),
            pltpu.VMEM((64, nvd * 16), jnp.float32),
            pltpu.VMEM((64, nvd * 16), jnp.float32),
            pltpu.VMEM((64, 16), jnp.float32),
            pltpu.VMEM((64, 16), jnp.float32),
            pltpu.VMEM((64, nvd * 16), jnp.float32),
            pltpu.VMEM((64, 16), jnp.float32),
            pltpu.SemaphoreType.DMA,
            pltpu.SemaphoreType.DMA,
            pltpu.VMEM_SHARED((NPAD, nvd * 16), jnp.float32),
            pltpu.VMEM_SHARED((NPAD, nvd * 16), jnp.float32),
            pltpu.VMEM_SHARED((NPAD, 16), jnp.float32),
        ],
    )
    def sc_kernel(tab_hbm, rows_hbm, src_hbm, dst_hbm, outd_hbm, outw_hbm,
                  tabv, srcv, dstv, idx0, idx1, dsti0, dsti1, aea0, aea1,
                  rowg0, rowg1, wden0, wden1, zbd, zbw, sem0, sem1,
                  rows_sp, accd, accw):
        c = lax.axis_index("c")
        s = lax.axis_index("s")
        idxv = (idx0, idx1)
        dstiv = (dsti0, dsti1)
        aeav = (aea0, aea1)
        rowg = (rowg0, rowg1)
        wden = (wden0, wden1)
        sem = (sem0, sem1)

        def _zb(e, _):
            for j in range(nvd):
                zbd[e, pl.ds(j * 16, 16)] = jnp.zeros((16,), jnp.float32)
            zbw[e] = jnp.zeros((16,), jnp.float32)
            return 0
        lax.fori_loop(0, 64, _zb, 0)

        for gi in range(n_passes):
            if split_edges:
                tg = 0
                outbase = c * NPAD
                ebase = (c * 16 + s) * ept
            else:
                tg = n_passes * c + gi
                outbase = tg * NPAD
                ebase = s * ept

            r0 = s * RPT
            for k in range(RPT // 64):
                pltpu.sync_copy(zbd, accd.at[pl.ds(r0 + k * 64, 64)])
                pltpu.sync_copy(zbw, accw.at[pl.ds(r0 + k * 64, 64)])
            rem = RPT % 64
            if rem:
                ro = r0 + (RPT // 64) * 64
                pltpu.sync_copy(zbd.at[pl.ds(0, rem)],
                                accd.at[pl.ds(ro, rem)])
                pltpu.sync_copy(zbw.at[pl.ds(0, rem)],
                                accw.at[pl.ds(ro, rem)])

            rowbase = tg * NPAD
            pltpu.sync_copy(tab_hbm.at[pl.ds(tg * (NPAD * 2), NPAD * 2)], tabv)
            pltpu.sync_copy(rows_hbm.at[pl.ds(rowbase + r0, RPT)],
                            rows_sp.at[pl.ds(r0, RPT)])
            plsc.subcore_barrier()

            def meta(h, b):
                def quad(q, _):
                    o = h * 64 + q * 16
                    s16 = srcv[pl.ds(o, 16)]
                    d16 = dstv[pl.ds(o, 16)]
                    asl = plsc.load_gather(tabv, [s16 * 2])
                    adl = plsc.load_gather(tabv, [d16 * 2 + 1])
                    a = asl + adl
                    a = jnp.where(a > 0, a, a * 0.2)
                    aeav[b][pl.ds(q * 16, 16)] = jnp.exp(a)
                    idxv[b][pl.ds(q * 16, 16)] = s16
                    dstiv[b][pl.ds(q * 16, 16)] = d16
                    return 0
                lax.fori_loop(0, 4, quad, 0)

            def gcopy(b):
                return pltpu.make_async_copy(rows_sp.at[idxv[b]], rowg[b],
                                             sem[b])

            def blk_body(blk, _):
                eoff = ebase + blk * CH
                pltpu.sync_copy(src_hbm.at[pl.ds(eoff, CH)], srcv)
                pltpu.sync_copy(dst_hbm.at[pl.ds(eoff, CH)], dstv)

                for b in range(2):
                    meta(b, b)
                    gcopy(b).start()

                def pair(p, _):
                    for b in range(2):
                        h = 2 * p + b
                        gcopy(b).wait()

                        def scale(q, _):
                            it = iota16()
                            av = aeav[b][pl.ds(q * 16, 16)]
                            for el in range(16):
                                e = q * 16 + el
                                sA = av[el]
                                for j in range(nvd):
                                    rowg[b][e, pl.ds(j * 16, 16)] = (
                                        rowg[b][e, pl.ds(j * 16, 16)] * sA)
                                wden[b][e] = jnp.where(it == 0, sA, 0.0)
                            return 0
                        lax.fori_loop(0, 4, scale, 0)

                        pltpu.sync_copy(rowg[b], accd.at[dstiv[b]], add=True)
                        pltpu.sync_copy(wden[b], accw.at[dstiv[b]], add=True)

                        @pl.when(h + 2 < HB)
                        def _():
                            meta(h + 2, b)
                            gcopy(b).start()
                    return 0
                lax.fori_loop(0, HB // 2, pair, 0)
                return 0
            lax.fori_loop(0, nblk, blk_body, 0)

            plsc.subcore_barrier()
            pltpu.sync_copy(accd.at[pl.ds(r0, RPT)],
                            outd_hbm.at[pl.ds(outbase + r0, RPT)])
            pltpu.sync_copy(accw.at[pl.ds(r0, RPT)],
                            outw_hbm.at[pl.ds(outbase + r0, RPT)])
            plsc.subcore_barrier()

    return sc_kernel


_sc_gat1 = _make_sc_gat(nvd=4, n_passes=4, split_edges=False)
_sc_gat2 = _make_sc_gat(nvd=2, n_passes=1, split_edges=True)


def kernel(x, edge_index, W1, att_src1, att_dst1, b1, W2, att_src2, att_dst2,
           b2, fcW, fcb):
    src = edge_index[0].astype(jnp.int32)
    dst = edge_index[1].astype(jnp.int32)
    loop = jnp.arange(N, dtype=jnp.int32)
    padv = jnp.full((EPAD - ESL,), N, dtype=jnp.int32)
    srcp = jnp.concatenate([src, loop, padv])
    dstp = jnp.concatenate([dst, loop, padv])

    xpad = jnp.concatenate(
        [x, jnp.zeros((NPAD - N, IN_DIM), jnp.float32)], axis=0)

    T1 = jnp.zeros((H1, H1 * C1, 2), jnp.float32)
    for hd in range(H1):
        T1 = T1.at[hd, hd * C1:(hd + 1) * C1, 0].set(att_src1[hd])
        T1 = T1.at[hd, hd * C1:(hd + 1) * C1, 1].set(att_dst1[hd])
    T2 = jnp.stack([att_src2[0], att_dst2[0]], axis=1)

    h1tab, tab1 = _tc_pre(xpad, W1, T1)
    out1d, out1w = _sc_gat1(tab1.reshape(H1 * NPAD * 2),
                            h1tab.reshape(H1 * NPAD, C1), srcp, dstp)
    h2tab, tab2 = _tc_mid(out1d.reshape(H1, NPAD, C1),
                          out1w.reshape(H1, NPAD, 16),
                          b1.reshape(1, H1 * C1), W2, T2)
    out2d, out2w = _sc_gat2(tab2.reshape(NPAD * 2), h2tab, srcp, dstp)
    emb, logits = _tc_post(out2d.reshape(2, NPAD, C2),
                           out2w.reshape(2, NPAD, 16),
                           b2.reshape(1, C2), fcW, fcb.reshape(1, OUT_DIM))
    return (emb, logits)
